# baseline (speedup 1.0000x reference)
import functools

import jax
import jax.numpy as jnp
from jax import lax
from jax.experimental import pallas as pl
from jax.experimental.pallas import tpu as pltpu
from jax.experimental.pallas import tpu_sc as plsc

F32 = jnp.float32
I32 = jnp.int32

GX, GY, GZ = 200, 200, 16
C = 16
SX = 2
NSLAB = GX // SX
NTILE = 32
CH = 64
VOXEL = 0.4
X0F, Y0F, Z0F = -40.0, -40.0, -1.0
EPSV = 1e-6

DENS_WORDS = SX * GY * GZ
FEAT_WORDS = DENS_WORDS * C


def _sc_voxelize(prm, bins, zeros_f):
    mesh = plsc.VectorSubcoreMesh(core_axis_name="c", subcore_axis_name="s")

    @functools.partial(
        pl.kernel,
        mesh=mesh,
        out_type=[
            jax.ShapeDtypeStruct((GX * GY * GZ,), F32),
            jax.ShapeDtypeStruct((GX * GY * GZ * C,), F32),
        ],
        scratch_types=[
            pltpu.VMEM((352,), I32),
            pltpu.VMEM((CH * 48,), F32),
            pltpu.VMEM((48,), F32),
            pltpu.VMEM((DENS_WORDS,), F32),
            pltpu.VMEM((FEAT_WORDS,), F32),
        ],
    )
    def vox(prm_hbm, bins_hbm, zf_hbm, dens_out,
            feat_out, bins_v, buf_v, prof_v, dens_s, feat_s):
        cid = lax.axis_index("c")
        sid = lax.axis_index("s")
        tid = sid * 2 + cid
        pltpu.sync_copy(bins_hbm, bins_v)
        lanes = lax.broadcasted_iota(I32, (16,), 0)
        lanesf = lanes.astype(F32)
        zcv = lanesf * VOXEL + (Z0F + 0.5 * VOXEL)

        lo_t = bins_v[pl.ds(256 + tid, 16)][0]
        hi_t = bins_v[pl.ds(288 + tid, 16)][0]

        def slab_body(slab, carry):
            if True:
                x0 = slab * SX
                pltpu.sync_copy(zf_hbm.at[pl.ds(0, DENS_WORDS)], dens_s)
                pltpu.sync_copy(zf_hbm, feat_s)
                g_lo = bins_v[pl.ds(jnp.maximum(x0 - 5, 0), 16)][0]
                g_hi = bins_v[pl.ds(x0 + SX, 16)][0]
                nch = (g_hi - g_lo + (CH - 1)) // CH

                def chunk_body(ci, ccarry):
                    base = g_lo + ci * CH
                    n = jnp.minimum(g_hi - base, CH)
                    boff = base * 48
                    pltpu.sync_copy(prm_hbm.at[pl.ds(boff, CH * 48)], buf_v)

                    def visit(vi, vcarry):
                        @pl.when(vi < n)
                        def _():
                            b16 = vi * 48
                            pir = buf_v[pl.ds(b16 + 16, 16)]

                            @pl.when(pir[1] >= x0.astype(F32))
                            def _():
                                _do_visit(b16, pir)

                        return vcarry

                    def _do_visit(b16, pir):
                            pfr = buf_v[pl.ds(b16, 16)]
                            mx = pfr[0]
                            my = pfr[1]
                            mz = pfr[2]
                            ax = pfr[3]
                            ay = pfr[4]
                            az = pfr[5]
                            wg = pfr[6]
                            loxf = pir[0]
                            hixf = pir[1]
                            loyf = pir[2]
                            hiyf = pir[3]
                            lozf = pir[4]
                            hizf = pir[5]
                            lox = loxf.astype(I32)
                            loy = loyf.astype(I32)
                            loz = lozf.astype(I32)
                            ftv = buf_v[pl.ds(b16 + 32, 16)]

                            dz = zcv - mz
                            ez = jnp.exp(-(dz * dz) * az)
                            ez = jnp.where(
                                (lanesf >= lozf) & (lanesf <= hizf), ez, 0.0)

                            ycv = (loyf + lanesf) * VOXEL + (
                                Y0F + 0.5 * VOXEL)
                            dy = ycv - my
                            ey = jnp.exp(-(dy * dy) * ay)
                            ey = jnp.where(lanesf <= (hiyf - loyf), ey, 0.0)

                            xcv = (loxf + lanesf) * VOXEL + (
                                X0F + 0.5 * VOXEL)
                            dx = xcv - mx
                            ex = jnp.exp(-(dx * dx) * ax) * wg
                            ex = jnp.where(lanesf <= (hixf - loxf), ex, 0.0)

                            prof_v[pl.ds(0, 16)] = ez
                            prof_v[pl.ds(16, 16)] = ex

                            span_z = hizf - lozf
                            pvecs = []
                            zvs = []
                            for kz in range(6):
                                zv = jnp.minimum(loz + kz, GZ - 1)
                                ezs = prof_v[pl.ds(zv, 16)][0]
                                ezs = jnp.where(kz <= span_z, ezs, 0.0)
                                pvecs.append(ftv * ezs)
                                zvs.append(zv)

                            for xl in range(SX):
                                wi = (x0 + xl) - lox
                                wic = jnp.clip(wi, 0, 15)
                                wxs = prof_v[pl.ds(16 + wic, 16)][0]
                                wxs = jnp.where(wi >= 0, wxs, 0.0)

                                @pl.when(wxs != 0.0)
                                def _(xl=xl, wxs=wxs):
                                    for ky in range(6):
                                        wys = ey[ky]
                                        wxy = wxs * wys
                                        yv = jnp.minimum(loy + ky, GY - 1)
                                        doff = (xl * GY + yv) * GZ
                                        plsc.addupdate(
                                            dens_s.at[pl.ds(doff, 16)],
                                            ez * wxy)
                                        foff = doff * C
                                        for kz in range(6):
                                            plsc.addupdate(
                                                feat_s.at[pl.ds(
                                                    foff + zvs[kz] * C, 16)],
                                                pvecs[kz] * wxy)

                    lax.fori_loop(0, CH, visit, 0)
                    return ccarry

                lax.fori_loop(0, nch, chunk_body, 0)

                def norm_cell(i, ncarry):
                    dv = dens_s[pl.ds(i * 16, 16)]
                    dc = jnp.maximum(dv, EPSV)
                    fbase = i * 256
                    for z in range(16):
                        off = fbase + z * 16
                        feat_s[pl.ds(off, 16)] = (
                            feat_s[pl.ds(off, 16)] / dc[z])
                    return ncarry

                lax.fori_loop(0, SX * GY, norm_cell, 0)
                pltpu.sync_copy(
                    dens_s, dens_out.at[pl.ds(x0 * GY * GZ, DENS_WORDS)])
                pltpu.sync_copy(
                    feat_s,
                    feat_out.at[pl.ds(x0 * GY * GZ * C, FEAT_WORDS)])

            return carry

        lax.fori_loop(lo_t, hi_t, slab_body, 0)

    return vox(prm, bins, zeros_f)


def kernel(means3d, opacities, covariances, features):
    N = means3d.shape[0]
    vol_min = jnp.array([X0F, Y0F, Z0F], F32)
    vol_max = jnp.array([40.0, 40.0, 5.4], F32)
    gs = jnp.array([GX, GY, GZ], I32)

    sig = jnp.sqrt(jnp.diagonal(covariances, axis1=1, axis2=2))
    lo_b = means3d - 3.0 * sig
    hi_b = means3d + 3.0 * sig
    valid = jnp.all(hi_b > vol_min, axis=1) & jnp.all(lo_b < vol_max, axis=1)
    lo_i = ((jnp.clip(lo_b, vol_min, vol_max) - vol_min) / VOXEL).astype(I32)
    hi_i = jnp.minimum(
        ((jnp.clip(hi_b, vol_min, vol_max) - vol_min) / VOXEL).astype(I32),
        gs - 1)
    w = opacities[:, 0] * valid.astype(F32)
    a = 0.5 / (sig * sig)

    keys, order = lax.sort_key_val(lo_i[:, 0], jnp.arange(N, dtype=I32))

    zcol = jnp.zeros((N,), F32)
    pf = jnp.stack(
        [means3d[:, 0], means3d[:, 1], means3d[:, 2], a[:, 0], a[:, 1],
         a[:, 2], w] + [zcol] * 9, axis=1)
    pi = jnp.stack(
        [lo_i[:, 0], hi_i[:, 0], lo_i[:, 1], hi_i[:, 1], lo_i[:, 2],
         hi_i[:, 2]] + [jnp.zeros((N,), I32)] * 10, axis=1)
    big = jnp.concatenate([pf, pi.astype(F32), features], axis=1)
    prm = jnp.concatenate([big[order], jnp.zeros((CH, 48), F32)]).reshape(-1)

    bins = jnp.searchsorted(keys, jnp.arange(201, dtype=I32),
                            side='left').astype(I32)
    bins = jnp.concatenate([bins, jnp.full((55,), N, I32)])
    x0s = jnp.arange(NSLAB, dtype=I32) * SX
    cost = bins[x0s + SX] - bins[jnp.maximum(x0s - 5, 0)]
    ccost = jnp.cumsum(cost)
    targets = (jnp.arange(1, NTILE) * ccost[-1]) // NTILE
    bnd = jnp.searchsorted(ccost, targets, side='left').astype(I32)
    tile_lo = jnp.concatenate([jnp.zeros((1,), I32), bnd])
    tile_hi = jnp.concatenate([bnd, jnp.full((1,), NSLAB, I32)])
    bins = jnp.concatenate([bins, tile_lo, tile_hi,
                            jnp.zeros((32,), I32)])
    zeros_f = jnp.zeros((FEAT_WORDS,), F32)

    dens_flat, feat_flat = _sc_voxelize(prm, bins, zeros_f)
    grid_density = dens_flat.reshape(GX, GY, GZ, 1)
    grid_feats = feat_flat.reshape(GX, GY, GZ, C)
    return grid_density, grid_feats


# speedup vs baseline: 1.0283x; 1.0283x over previous
import functools

import jax
import jax.numpy as jnp
from jax import lax
from jax.experimental import pallas as pl
from jax.experimental.pallas import tpu as pltpu
from jax.experimental.pallas import tpu_sc as plsc

F32 = jnp.float32
I32 = jnp.int32

GX, GY, GZ = 200, 200, 16
C = 16
SX = 2
NSLAB = GX // SX
NTILE = 32
CH = 64
VOXEL = 0.4
X0F, Y0F, Z0F = -40.0, -40.0, -1.0
EPSV = 1e-6

DENS_WORDS = SX * GY * GZ
FEAT_WORDS = DENS_WORDS * C


def _sc_voxelize(prm, bins, zeros_f):
    mesh = plsc.VectorSubcoreMesh(core_axis_name="c", subcore_axis_name="s")

    @functools.partial(
        pl.kernel,
        mesh=mesh,
        out_type=[
            jax.ShapeDtypeStruct((GX * GY * GZ,), F32),
            jax.ShapeDtypeStruct((GX * GY * GZ * C,), F32),
        ],
        scratch_types=[
            pltpu.VMEM((352,), I32),
            pltpu.VMEM((CH * 48,), F32),
            pltpu.VMEM((CH * 48,), F32),
            pltpu.VMEM((48,), F32),
            pltpu.VMEM((DENS_WORDS,), F32),
            pltpu.VMEM((FEAT_WORDS,), F32),
            pltpu.SemaphoreType.DMA,
            pltpu.SemaphoreType.DMA,
        ],
    )
    def vox(prm_hbm, bins_hbm, zf_hbm, dens_out, feat_out, bins_v,
            buf_v, buf2_v, prof_v, dens_s, feat_s, sem0, sem1):
        cid = lax.axis_index("c")
        sid = lax.axis_index("s")
        tid = sid * 2 + cid
        pltpu.sync_copy(bins_hbm, bins_v)
        lanes = lax.broadcasted_iota(I32, (16,), 0)
        lanesf = lanes.astype(F32)
        zcv = lanesf * VOXEL + (Z0F + 0.5 * VOXEL)

        lo_t = bins_v[pl.ds(256 + tid, 16)][0]
        hi_t = bins_v[pl.ds(288 + tid, 16)][0]

        def slab_body(slab, carry):
            if True:
                x0 = slab * SX
                pltpu.sync_copy(zf_hbm.at[pl.ds(0, DENS_WORDS)], dens_s)
                pltpu.sync_copy(zf_hbm, feat_s)
                g_lo = bins_v[pl.ds(jnp.maximum(x0 - 5, 0), 16)][0]
                g_hi = bins_v[pl.ds(x0 + SX, 16)][0]
                nch = (g_hi - g_lo + (CH - 1)) // CH

                def start_fetch(ci, buf, sem):
                    boff = (g_lo + ci * CH) * 48
                    pltpu.async_copy(
                        prm_hbm.at[pl.ds(boff, CH * 48)], buf, sem)

                def wait_fetch(buf, sem):
                    pltpu.make_async_copy(
                        prm_hbm.at[pl.ds(0, CH * 48)], buf, sem).wait()

                def process(ci, buf):
                    n = jnp.minimum(g_hi - (g_lo + ci * CH), CH)

                    def visit(vi, vcarry):
                        @pl.when(vi < n)
                        def _():
                            b16 = vi * 48
                            pir = buf[pl.ds(b16 + 16, 16)]

                            @pl.when(pir[1] >= x0.astype(F32))
                            def _():
                                _do_visit(buf, b16, pir)

                        return vcarry

                    lax.fori_loop(0, CH, visit, 0)

                def _do_visit(buf, b16, pir):
                            pfr = buf[pl.ds(b16, 16)]
                            mx = pfr[0]
                            my = pfr[1]
                            mz = pfr[2]
                            ax = pfr[3]
                            ay = pfr[4]
                            az = pfr[5]
                            wg = pfr[6]
                            loxf = pir[0]
                            hixf = pir[1]
                            loyf = pir[2]
                            hiyf = pir[3]
                            lozf = pir[4]
                            hizf = pir[5]
                            lox = loxf.astype(I32)
                            loy = loyf.astype(I32)
                            loz = lozf.astype(I32)
                            ftv = buf[pl.ds(b16 + 32, 16)]

                            dz = zcv - mz
                            ez = jnp.exp(-(dz * dz) * az)
                            ez = jnp.where(
                                (lanesf >= lozf) & (lanesf <= hizf), ez, 0.0)

                            ycv = (loyf + lanesf) * VOXEL + (
                                Y0F + 0.5 * VOXEL)
                            dy = ycv - my
                            ey = jnp.exp(-(dy * dy) * ay)
                            ey = jnp.where(lanesf <= (hiyf - loyf), ey, 0.0)

                            xcv = (loxf + lanesf) * VOXEL + (
                                X0F + 0.5 * VOXEL)
                            dx = xcv - mx
                            ex = jnp.exp(-(dx * dx) * ax) * wg
                            ex = jnp.where(lanesf <= (hixf - loxf), ex, 0.0)

                            prof_v[pl.ds(0, 16)] = ez
                            prof_v[pl.ds(16, 16)] = ex

                            span_z = hizf - lozf
                            pvecs = []
                            zvs = []
                            for kz in range(6):
                                zv = jnp.minimum(loz + kz, GZ - 1)
                                ezs = prof_v[pl.ds(zv, 16)][0]
                                ezs = jnp.where(kz <= span_z, ezs, 0.0)
                                pvecs.append(ftv * ezs)
                                zvs.append(zv)

                            for xl in range(SX):
                                wi = (x0 + xl) - lox
                                wic = jnp.clip(wi, 0, 15)
                                wxs = prof_v[pl.ds(16 + wic, 16)][0]
                                wxs = jnp.where(wi >= 0, wxs, 0.0)

                                @pl.when(wxs != 0.0)
                                def _(xl=xl, wxs=wxs):
                                    for ky in range(6):
                                        wys = ey[ky]
                                        wxy = wxs * wys
                                        yv = jnp.minimum(loy + ky, GY - 1)
                                        doff = (xl * GY + yv) * GZ
                                        plsc.addupdate(
                                            dens_s.at[pl.ds(doff, 16)],
                                            ez * wxy)
                                        foff = doff * C
                                        for kz in range(6):
                                            plsc.addupdate(
                                                feat_s.at[pl.ds(
                                                    foff + zvs[kz] * C, 16)],
                                                pvecs[kz] * wxy)

                @pl.when(nch > 0)
                def _():
                    start_fetch(0, buf_v, sem0)

                def pair_body(k, pcarry):
                    i0 = 2 * k
                    i1 = i0 + 1
                    wait_fetch(buf_v, sem0)

                    @pl.when(i1 < nch)
                    def _():
                        start_fetch(i1, buf2_v, sem1)

                    process(i0, buf_v)

                    @pl.when(i1 < nch)
                    def _():
                        wait_fetch(buf2_v, sem1)

                        @pl.when(i1 + 1 < nch)
                        def _():
                            start_fetch(i1 + 1, buf_v, sem0)

                        process(i1, buf2_v)

                    return pcarry

                lax.fori_loop(0, (nch + 1) // 2, pair_body, 0)

                def norm_cell(i, ncarry):
                    dv = dens_s[pl.ds(i * 16, 16)]
                    dc = jnp.maximum(dv, EPSV)
                    fbase = i * 256
                    for z in range(16):
                        off = fbase + z * 16
                        feat_s[pl.ds(off, 16)] = (
                            feat_s[pl.ds(off, 16)] / dc[z])
                    return ncarry

                lax.fori_loop(0, SX * GY, norm_cell, 0)
                pltpu.sync_copy(
                    dens_s, dens_out.at[pl.ds(x0 * GY * GZ, DENS_WORDS)])
                pltpu.sync_copy(
                    feat_s,
                    feat_out.at[pl.ds(x0 * GY * GZ * C, FEAT_WORDS)])

            return carry

        lax.fori_loop(lo_t, hi_t, slab_body, 0)

    return vox(prm, bins, zeros_f)


def kernel(means3d, opacities, covariances, features):
    N = means3d.shape[0]
    vol_min = jnp.array([X0F, Y0F, Z0F], F32)
    vol_max = jnp.array([40.0, 40.0, 5.4], F32)
    gs = jnp.array([GX, GY, GZ], I32)

    sig = jnp.sqrt(jnp.diagonal(covariances, axis1=1, axis2=2))
    lo_b = means3d - 3.0 * sig
    hi_b = means3d + 3.0 * sig
    valid = jnp.all(hi_b > vol_min, axis=1) & jnp.all(lo_b < vol_max, axis=1)
    lo_i = ((jnp.clip(lo_b, vol_min, vol_max) - vol_min) / VOXEL).astype(I32)
    hi_i = jnp.minimum(
        ((jnp.clip(hi_b, vol_min, vol_max) - vol_min) / VOXEL).astype(I32),
        gs - 1)
    w = opacities[:, 0] * valid.astype(F32)
    a = 0.5 / (sig * sig)

    keys, order = lax.sort_key_val(lo_i[:, 0], jnp.arange(N, dtype=I32))

    zcol = jnp.zeros((N,), F32)
    pf = jnp.stack(
        [means3d[:, 0], means3d[:, 1], means3d[:, 2], a[:, 0], a[:, 1],
         a[:, 2], w] + [zcol] * 9, axis=1)
    pi = jnp.stack(
        [lo_i[:, 0], hi_i[:, 0], lo_i[:, 1], hi_i[:, 1], lo_i[:, 2],
         hi_i[:, 2]] + [jnp.zeros((N,), I32)] * 10, axis=1)
    big = jnp.concatenate([pf, pi.astype(F32), features], axis=1)
    prm = jnp.concatenate([big[order], jnp.zeros((CH, 48), F32)]).reshape(-1)

    bins = jnp.searchsorted(keys, jnp.arange(201, dtype=I32),
                            side='left').astype(I32)
    bins = jnp.concatenate([bins, jnp.full((55,), N, I32)])
    x0s = jnp.arange(NSLAB, dtype=I32) * SX
    cost = bins[x0s + SX] - bins[jnp.maximum(x0s - 5, 0)]
    ccost = jnp.cumsum(cost)
    targets = (jnp.arange(1, NTILE) * ccost[-1]) // NTILE
    bnd = jnp.searchsorted(ccost, targets, side='left').astype(I32)
    tile_lo = jnp.concatenate([jnp.zeros((1,), I32), bnd])
    tile_hi = jnp.concatenate([bnd, jnp.full((1,), NSLAB, I32)])
    bins = jnp.concatenate([bins, tile_lo, tile_hi,
                            jnp.zeros((32,), I32)])
    zeros_f = jnp.zeros((FEAT_WORDS,), F32)

    dens_flat, feat_flat = _sc_voxelize(prm, bins, zeros_f)
    grid_density = dens_flat.reshape(GX, GY, GZ, 1)
    grid_feats = feat_flat.reshape(GX, GY, GZ, C)
    return grid_density, grid_feats
